# baseline (speedup 1.0000x reference)
import jax
import jax.numpy as jnp
from jax import lax
from jax.experimental import pallas as pl
from jax.experimental.pallas import tpu as pltpu
from jax.experimental.pallas import tpu_sc as plsc

EMB = 64
B, LSEQ, T = 1024, 50, 20
NC, NS, LANES = 2, 16, 16
NW = NC * NS
NE = EMB // LANES
G = 25
BLKS = (B * LSEQ) // G
BPW = BLKS // NW
NPAIR = BPW // 2
IDX_PER_BLK = G * T
IDX_CHUNK = 125
NSUB = IDX_PER_BLK // IDX_CHUNK


def _sc_body(seq_hbm, w_hbm, out_hbm,
             idx0, idx1, rows0, rows1, out0, out1,
             sem0, sem1, osem0, osem1):
    wid = lax.axis_index("s") * NC + lax.axis_index("c")

    bufs = ((idx0, rows0, out0, sem0, osem0),
            (idx1, rows1, out1, sem1, osem1))

    def stage(blk, buf):
        idx_v, rows_v, _, sem, _ = bufs[buf]
        pltpu.sync_copy(seq_hbm.at[blk], idx_v)
        for j in range(NSUB):
            pltpu.async_copy(
                w_hbm.at[idx_v.at[j]],
                rows_v.at[pl.ds(j * IDX_CHUNK, IDX_CHUNK)],
                sem,
            )

    def drain(buf):
        idx_v, rows_v, _, sem, _ = bufs[buf]
        for j in range(NSUB):
            pltpu.make_async_copy(
                w_hbm.at[idx_v.at[j]],
                rows_v.at[pl.ds(j * IDX_CHUNK, IDX_CHUNK)],
                sem,
            ).wait()

    def compute(blk, buf, first):
        idx_v, rows_v, out_v, _, osem = bufs[buf]
        b = blk >> 1
        half = blk & 1

        @pl.when(jnp.logical_not(first))
        def _():
            pltpu.make_async_copy(
                out_v, out_hbm.at[b - 1, pl.ds(half * G, G)], osem
            ).wait()

        @plsc.parallel_loop(0, G, 1, unroll=2)
        def group(g):
            base = g * T
            s = [jnp.zeros((LANES,), jnp.float32) for _ in range(NE)]
            c = [jnp.zeros((LANES,), jnp.int32) for _ in range(NE)]
            for t in range(T):
                for e in range(NE):
                    r = rows_v[base + t, pl.ds(e * LANES, LANES)]
                    s[e] = s[e] + r
                    bb = lax.bitcast_convert_type(r, jnp.int32)
                    c[e] = jnp.where(bb != 0, c[e] + 1, c[e])
            for e in range(NE):
                cf = c[e].astype(jnp.float32)
                out_v[g, pl.ds(e * LANES, LANES)] = jnp.where(
                    c[e] == 0, 0.0, s[e] / cf)

        pltpu.async_copy(out_v, out_hbm.at[b, pl.ds(half * G, G)], osem)

    stage(wid * BPW, 0)
    stage(wid * BPW + 1, 1)

    def outer(io, carry):
        blk = wid * BPW + 2 * io
        drain(0)
        compute(blk, 0, first=io == 0)

        @pl.when(io < NPAIR - 1)
        def _():
            stage(blk + 2, 0)

        drain(1)
        compute(blk + 1, 1, first=io == 0)

        @pl.when(io < NPAIR - 1)
        def _():
            stage(blk + 3, 1)

        return carry

    lax.fori_loop(0, NPAIR, outer, 0)
    for buf in range(2):
        _, _, out_v, _, osem = bufs[buf]
        last = wid * BPW + BPW - 2 + buf
        pltpu.make_async_copy(
            out_v,
            out_hbm.at[last >> 1, pl.ds((last & 1) * G, G)],
            osem,
        ).wait()


def kernel(seq, W):
    w_full = jnp.pad(W, ((1, 0), (0, 0)))
    seq3 = seq.reshape(BLKS, NSUB, IDX_CHUNK)
    mesh = plsc.VectorSubcoreMesh(core_axis_name="c", subcore_axis_name="s")
    out = pl.kernel(
        _sc_body,
        mesh=mesh,
        compiler_params=pltpu.CompilerParams(use_tc_tiling_on_sc=False),
        out_type=jax.ShapeDtypeStruct((B, LSEQ, EMB), jnp.float32),
        scratch_types=[
            pltpu.VMEM((NSUB, IDX_CHUNK), jnp.int32),
            pltpu.VMEM((NSUB, IDX_CHUNK), jnp.int32),
            pltpu.VMEM((IDX_PER_BLK, EMB), jnp.float32),
            pltpu.VMEM((IDX_PER_BLK, EMB), jnp.float32),
            pltpu.VMEM((G, EMB), jnp.float32),
            pltpu.VMEM((G, EMB), jnp.float32),
            pltpu.SemaphoreType.DMA,
            pltpu.SemaphoreType.DMA,
            pltpu.SemaphoreType.DMA,
            pltpu.SemaphoreType.DMA,
        ],
    )(seq3, w_full)
    return out
